# baseline (speedup 1.0000x reference)
import functools

import jax
import jax.numpy as jnp
from jax import lax
from jax.experimental import pallas as pl
from jax.experimental.pallas import tpu as pltpu
from jax.experimental.pallas import tpu_sc as plsc

_DIM = 64
_PDIM = 128
_V = 1000000
_BATCH, _FIELDS = 16384, 26
_B = _BATCH * _FIELDS
_NC, _NS = 2, 16
_NW = _NC * _NS

_mesh = plsc.VectorSubcoreMesh(core_axis_name="c", subcore_axis_name="s")
_params_t = pltpu.CompilerParams(use_tc_tiling_on_sc=True,
                                 needs_layout_passes=False)
_params_g = pltpu.CompilerParams(use_tc_tiling_on_sc=True)


_NPAN = _V // _PDIM
_VTAIL = _V - _NPAN * _PDIM
_ITERS = (_NPAN + _NW - 1) // _NW


@functools.partial(
    pl.kernel,
    mesh=_mesh,
    compiler_params=_params_t,
    out_type=jax.ShapeDtypeStruct((_V, _PDIM), jnp.float32),
    scratch_types=[
        pltpu.VMEM((_DIM, _PDIM), jnp.float32),
        pltpu.VMEM((_DIM, _PDIM), jnp.float32),
        pltpu.VMEM((_PDIM, _PDIM), jnp.float32),
        pltpu.VMEM((_PDIM, _PDIM), jnp.float32),
    ] + [pltpu.SemaphoreType.DMA] * 4,
)
def _transpose_kernel(feat_hbm, tail_hbm, tab_hbm, inA, inB, outA, outB,
                      rsemA, rsemB, wsemA, wsemB):
    wid = lax.axis_index("s") * _NC + lax.axis_index("c")
    c_vecs = [lax.iota(jnp.int32, 16) + 16 * k for k in range(4)]

    def panel_of(i):
        return i * _NW + wid

    def fire_read(i, ibuf, rsem):
        pltpu.async_copy(
            feat_hbm.at[:, pl.ds(panel_of(i) * _PDIM, _PDIM)], ibuf, rsem)

    def wait_read(ibuf, rsem):
        pltpu.make_async_copy(
            feat_hbm.at[:, pl.ds(0, _PDIM)], ibuf, rsem).wait()

    def fire_write(i, obuf, wsem):
        pltpu.async_copy(
            obuf, tab_hbm.at[pl.ds(panel_of(i) * _PDIM, _PDIM)], wsem)

    def drain_write(obuf, wsem):
        pltpu.make_async_copy(
            obuf, tab_hbm.at[pl.ds(0, _PDIM)], wsem).wait()

    def compute(ibuf, obuf, nrows):
        def tr_body(g, carry):
            for u in range(8):
                r = g * 8 + u
                r_vec = jnp.full((16,), 0, jnp.int32) + r
                for k in range(4):
                    v = plsc.load_gather(ibuf, [c_vecs[k], r_vec])
                    plsc.store_scatter(obuf, [r_vec, c_vecs[k]], v)
            return carry
        lax.fori_loop(0, nrows // 8, tr_body, 0)

    def process(i, ibuf, obuf, rsem, wsem):
        wait_read(ibuf, rsem)

        @pl.when(i >= 2)
        def _():
            drain_write(obuf, wsem)

        compute(ibuf, obuf, _PDIM)
        fire_write(i, obuf, wsem)

        @pl.when(panel_of(i + 2) < _NPAN)
        def _():
            fire_read(i + 2, ibuf, rsem)

    fire_read(0, inA, rsemA)
    fire_read(1, inB, rsemB)

    def pair_body(g, carry):
        for p in range(2):
            i = 2 * g + p
            ibuf, obuf = (inA, outA) if p == 0 else (inB, outB)
            rsem, wsem = (rsemA, wsemA) if p == 0 else (rsemB, wsemB)

            @pl.when(jnp.logical_and(i < _ITERS, panel_of(i) < _NPAN))
            def _():
                process(i, ibuf, obuf, rsem, wsem)
        return carry

    lax.fori_loop(0, (_ITERS + 1) // 2, pair_body, 0)

    drain_write(outA, wsemA)
    drain_write(outB, wsemB)

    @pl.when(wid == _NW - 1)
    def _():
        pltpu.async_copy(tail_hbm, outA, rsemA)
        pltpu.make_async_copy(tail_hbm, outA, rsemA).wait()
        pltpu.async_copy(
            outA.at[pl.ds(0, _VTAIL)],
            tab_hbm.at[pl.ds(_V - _VTAIL, _VTAIL)], wsemA)
        pltpu.make_async_copy(
            outA.at[pl.ds(0, _VTAIL)],
            tab_hbm.at[pl.ds(0, _VTAIL)], wsemA).wait()


_BPW = _B // _NW
_CHB = 4
_SUB = _CHB * _FIELDS
_NSUBT = _BPW // _SUB
_R = 8
_G = 5
_NOUT = _NSUBT // _R


@functools.partial(
    pl.kernel,
    mesh=_mesh,
    compiler_params=_params_g,
    out_type=jax.ShapeDtypeStruct((_BATCH, _FIELDS, _PDIM), jnp.float32),
    scratch_types=[
        pltpu.VMEM((_BPW,), jnp.int32),
        pltpu.VMEM((_R * _SUB, _PDIM), jnp.float32),
    ] + [pltpu.SemaphoreType.DMA] * (2 * _R),
)
def _gather_kernel(table_hbm, idx_hbm, out_hbm, idx_v, ring, *sems):
    gsems = sems[:_R]
    ssems = sems[_R:]
    wid = lax.axis_index("s") * _NC + lax.axis_index("c")
    base = wid * _BPW
    bbase = wid * (_BPW // _FIELDS)
    pltpu.sync_copy(idx_hbm.at[pl.ds(base, _BPW)], idx_v)

    def slot(s):
        return ring.at[pl.ds(s * _SUB, _SUB)]

    def fire_gather(i, s):
        idx_sl = idx_v.at[pl.ds(i * _SUB, _SUB)]
        pltpu.async_copy(table_hbm.at[idx_sl], slot(s), gsems[s])

    def wait_gather(s):
        pltpu.make_async_copy(
            table_hbm.at[pl.ds(0, _SUB)], slot(s), gsems[s]).wait()

    def fire_store(i, s):
        for m in range(_CHB):
            src = ring.at[pl.ds(s * _SUB + m * _FIELDS, _FIELDS)]
            pltpu.async_copy(src, out_hbm.at[bbase + i * _CHB + m], ssems[s])

    def drain_store(s):
        for m in range(_CHB):
            src = ring.at[pl.ds(s * _SUB + m * _FIELDS, _FIELDS)]
            pltpu.make_async_copy(src, out_hbm.at[bbase], ssems[s]).wait()

    for i in range(_G):
        fire_gather(i, i)

    def outer_body(g, carry):
        i0 = g * _R
        for s in range(_R):
            i = i0 + s

            @pl.when(i >= _R - _G)
            def _():
                drain_store((s + _G) % _R)

            @pl.when(i + _G < _NSUBT)
            def _():
                fire_gather(i + _G, (s + _G) % _R)

            wait_gather(s)
            fire_store(i, s)
        return carry

    lax.fori_loop(0, _NOUT, outer_body, 0)

    for j in range(_NSUBT - (_R - _G), _NSUBT):
        drain_store(j % _R)


def kernel(idxs, features):
    flat = idxs.reshape(-1).astype(jnp.int32)
    feat_t = jnp.transpose(features)
    tail = jnp.pad(features[_NPAN * _PDIM:, :],
                   ((0, _PDIM - _VTAIL), (0, _PDIM - _DIM)))
    table = _transpose_kernel(feat_t, tail)
    padded = _gather_kernel(table, flat)
    return padded[:, :, :_DIM]


# speedup vs baseline: 1.1959x; 1.1959x over previous
import functools

import jax
import jax.numpy as jnp
from jax import lax
from jax.experimental import pallas as pl
from jax.experimental.pallas import tpu as pltpu
from jax.experimental.pallas import tpu_sc as plsc

_DIM = 64
_PDIM = 128
_V = 1000000
_BATCH, _FIELDS = 16384, 26
_B = _BATCH * _FIELDS
_NC, _NS = 2, 16
_NW = _NC * _NS

_mesh = plsc.VectorSubcoreMesh(core_axis_name="c", subcore_axis_name="s")
_params_t = pltpu.CompilerParams(use_tc_tiling_on_sc=True,
                                 needs_layout_passes=False)
_params_g = pltpu.CompilerParams(use_tc_tiling_on_sc=True)


_NPAN = _V // _PDIM
_VTAIL = _V - _NPAN * _PDIM
_ITERS = (_NPAN + _NW - 1) // _NW


@functools.partial(
    pl.kernel,
    mesh=_mesh,
    compiler_params=_params_t,
    out_type=jax.ShapeDtypeStruct((_V, _PDIM), jnp.float32),
    scratch_types=[
        pltpu.VMEM((_DIM, _PDIM), jnp.float32),
        pltpu.VMEM((_DIM, _PDIM), jnp.float32),
        pltpu.VMEM((_PDIM, _PDIM), jnp.float32),
        pltpu.VMEM((_PDIM, _PDIM), jnp.float32),
    ] + [pltpu.SemaphoreType.DMA] * 4,
)
def _transpose_kernel(feat_hbm, tail_hbm, tab_hbm, inA, inB, outA, outB,
                      rsemA, rsemB, wsemA, wsemB):
    wid = lax.axis_index("s") * _NC + lax.axis_index("c")
    c_vecs = [lax.iota(jnp.int32, 16) + 16 * k for k in range(4)]

    def panel_of(i):
        return i * _NW + wid

    def fire_read(i, ibuf, rsem):
        pltpu.async_copy(
            feat_hbm.at[:, pl.ds(panel_of(i) * _PDIM, _PDIM)], ibuf, rsem)

    def wait_read(ibuf, rsem):
        pltpu.make_async_copy(
            feat_hbm.at[:, pl.ds(0, _PDIM)], ibuf, rsem).wait()

    def fire_write(i, obuf, wsem):
        pltpu.async_copy(
            obuf, tab_hbm.at[pl.ds(panel_of(i) * _PDIM, _PDIM)], wsem)

    def drain_write(obuf, wsem):
        pltpu.make_async_copy(
            obuf, tab_hbm.at[pl.ds(0, _PDIM)], wsem).wait()

    def compute(ibuf, obuf, nrows):
        r_vecs = [lax.iota(jnp.int32, 16) + 16 * j for j in range(8)]

        def ch_body(c, carry):
            c_vec = jnp.full((16,), 0, jnp.int32) + c
            row = ibuf.at[c]
            for j in range(nrows // 16):
                v = row[pl.ds(16 * j, 16)]
                plsc.store_scatter(obuf, [r_vecs[j], c_vec], v)
            return carry
        lax.fori_loop(0, _DIM, ch_body, 0)

    def process(i, ibuf, obuf, rsem, wsem):
        wait_read(ibuf, rsem)

        @pl.when(i >= 2)
        def _():
            drain_write(obuf, wsem)

        compute(ibuf, obuf, _PDIM)
        fire_write(i, obuf, wsem)

        @pl.when(panel_of(i + 2) < _NPAN)
        def _():
            fire_read(i + 2, ibuf, rsem)

    fire_read(0, inA, rsemA)
    fire_read(1, inB, rsemB)

    def pair_body(g, carry):
        for p in range(2):
            i = 2 * g + p
            ibuf, obuf = (inA, outA) if p == 0 else (inB, outB)
            rsem, wsem = (rsemA, wsemA) if p == 0 else (rsemB, wsemB)

            @pl.when(jnp.logical_and(i < _ITERS, panel_of(i) < _NPAN))
            def _():
                process(i, ibuf, obuf, rsem, wsem)
        return carry

    lax.fori_loop(0, (_ITERS + 1) // 2, pair_body, 0)

    drain_write(outA, wsemA)
    drain_write(outB, wsemB)

    @pl.when(wid == _NW - 1)
    def _():
        pltpu.async_copy(tail_hbm, outA, rsemA)
        pltpu.make_async_copy(tail_hbm, outA, rsemA).wait()
        pltpu.async_copy(
            outA.at[pl.ds(0, _VTAIL)],
            tab_hbm.at[pl.ds(_V - _VTAIL, _VTAIL)], wsemA)
        pltpu.make_async_copy(
            outA.at[pl.ds(0, _VTAIL)],
            tab_hbm.at[pl.ds(0, _VTAIL)], wsemA).wait()


_BPW = _B // _NW
_CHB = 4
_SUB = _CHB * _FIELDS
_NSUBT = _BPW // _SUB
_R = 8
_G = 5
_NOUT = _NSUBT // _R


@functools.partial(
    pl.kernel,
    mesh=_mesh,
    compiler_params=_params_g,
    out_type=jax.ShapeDtypeStruct((_BATCH, _FIELDS, _PDIM), jnp.float32),
    scratch_types=[
        pltpu.VMEM((_BPW,), jnp.int32),
        pltpu.VMEM((_R * _SUB, _PDIM), jnp.float32),
    ] + [pltpu.SemaphoreType.DMA] * (2 * _R),
)
def _gather_kernel(table_hbm, idx_hbm, out_hbm, idx_v, ring, *sems):
    gsems = sems[:_R]
    ssems = sems[_R:]
    wid = lax.axis_index("s") * _NC + lax.axis_index("c")
    base = wid * _BPW
    bbase = wid * (_BPW // _FIELDS)
    pltpu.sync_copy(idx_hbm.at[pl.ds(base, _BPW)], idx_v)

    def slot(s):
        return ring.at[pl.ds(s * _SUB, _SUB)]

    def fire_gather(i, s):
        idx_sl = idx_v.at[pl.ds(i * _SUB, _SUB)]
        pltpu.async_copy(table_hbm.at[idx_sl], slot(s), gsems[s])

    def wait_gather(s):
        pltpu.make_async_copy(
            table_hbm.at[pl.ds(0, _SUB)], slot(s), gsems[s]).wait()

    def fire_store(i, s):
        for m in range(_CHB):
            src = ring.at[pl.ds(s * _SUB + m * _FIELDS, _FIELDS)]
            pltpu.async_copy(src, out_hbm.at[bbase + i * _CHB + m], ssems[s])

    def drain_store(s):
        for m in range(_CHB):
            src = ring.at[pl.ds(s * _SUB + m * _FIELDS, _FIELDS)]
            pltpu.make_async_copy(src, out_hbm.at[bbase], ssems[s]).wait()

    for i in range(_G):
        fire_gather(i, i)

    def outer_body(g, carry):
        i0 = g * _R
        for s in range(_R):
            i = i0 + s

            @pl.when(i >= _R - _G)
            def _():
                drain_store((s + _G) % _R)

            @pl.when(i + _G < _NSUBT)
            def _():
                fire_gather(i + _G, (s + _G) % _R)

            wait_gather(s)
            fire_store(i, s)
        return carry

    lax.fori_loop(0, _NOUT, outer_body, 0)

    for j in range(_NSUBT - (_R - _G), _NSUBT):
        drain_store(j % _R)


def kernel(idxs, features):
    flat = idxs.reshape(-1).astype(jnp.int32)
    feat_t = jnp.transpose(features)
    tail = jnp.pad(features[_NPAN * _PDIM:, :],
                   ((0, _PDIM - _VTAIL), (0, _PDIM - _DIM)))
    table = _transpose_kernel(feat_t, tail)
    padded = _gather_kernel(table, flat)
    return padded[:, :, :_DIM]


# speedup vs baseline: 1.8279x; 1.5285x over previous
import functools

import jax
import jax.numpy as jnp
from jax import lax
from jax.experimental import pallas as pl
from jax.experimental.pallas import tpu as pltpu
from jax.experimental.pallas import tpu_sc as plsc

_DIM = 64
_B = 16384 * 26
_NC, _NS = 2, 16
_NW = _NC * _NS
_BPW = _B // _NW
_SUB = 128
_NSUBT = _BPW // _SUB
_R = 13
_G = 8
_NOUT = _NSUBT // _R

_mesh = plsc.VectorSubcoreMesh(core_axis_name="c", subcore_axis_name="s")


@functools.partial(
    pl.kernel,
    mesh=_mesh,
    compiler_params=pltpu.CompilerParams(use_tc_tiling_on_sc=False),
    out_type=jax.ShapeDtypeStruct((_B, _DIM), jnp.float32),
    scratch_types=[
        pltpu.VMEM((_BPW,), jnp.int32),
        pltpu.VMEM((_R * _SUB, _DIM), jnp.float32),
    ] + [pltpu.SemaphoreType.DMA] * (2 * _R),
)
def _gather_kernel(table_hbm, idx_hbm, out_hbm, idx_v, ring, *sems):
    gsems = sems[:_R]
    ssems = sems[_R:]
    wid = lax.axis_index("s") * _NC + lax.axis_index("c")
    base = wid * _BPW
    pltpu.sync_copy(idx_hbm.at[pl.ds(base, _BPW)], idx_v)

    def slot(s):
        return ring.at[pl.ds(s * _SUB, _SUB)]

    def fire_gather(i, s):
        idx_sl = idx_v.at[pl.ds(i * _SUB, _SUB)]
        pltpu.async_copy(table_hbm.at[idx_sl], slot(s), gsems[s])

    def wait_gather(s):
        pltpu.make_async_copy(
            table_hbm.at[pl.ds(0, _SUB)], slot(s), gsems[s]).wait()

    def fire_store(i, s):
        pltpu.async_copy(slot(s), out_hbm.at[pl.ds(base + i * _SUB, _SUB)],
                         ssems[s])

    def drain_store(s):
        pltpu.make_async_copy(
            slot(s), out_hbm.at[pl.ds(base, _SUB)], ssems[s]).wait()

    for i in range(_G):
        fire_gather(i, i)

    def outer_body(g, carry):
        i0 = g * _R
        for s in range(_R):
            i = i0 + s
            @pl.when(i >= _R - _G)
            def _():
                drain_store((s + _G) % _R)

            @pl.when(i + _G < _NSUBT)
            def _():
                fire_gather(i + _G, (s + _G) % _R)

            wait_gather(s)
            fire_store(i, s)
        return carry

    lax.fori_loop(0, _NOUT, outer_body, 0)

    for j in range(_NSUBT - (_R - _G), _NSUBT):
        drain_store(j % _R)


def kernel(idxs, features):
    flat = idxs.reshape(-1).astype(jnp.int32)
    out = _gather_kernel(features, flat)
    return out.reshape(idxs.shape + (features.shape[1],))
